# baseline (speedup 1.0000x reference)
import functools

import jax
import jax.numpy as jnp
from jax import lax
from jax.experimental import pallas as pl
from jax.experimental.pallas import tpu as pltpu
from jax.experimental.pallas import tpu_sc as plsc


def _prologue_body(eflo_ref, efhi_ref, x_ref, w_ref, b_ref, out_ref, pf_ref, pt_ref):
    de = eflo_ref.shape[-1]
    d = x_ref.shape[-1]
    w = w_ref[w_ref.shape[0] - de :, :]

    def pack(ef_ref):
        ep = jnp.dot(ef_ref[...], w, preferred_element_type=jnp.float32) + b_ref[...]
        bits = lax.bitcast_convert_type(ep.astype(jnp.bfloat16), jnp.uint16)
        half = ep.shape[-1] // 2
        lo = bits[:, :half].astype(jnp.uint32)
        hi = bits[:, half:].astype(jnp.uint32)
        return lax.bitcast_convert_type(lo | (hi << 16), jnp.int32)

    out_ref[...] = jnp.concatenate([pack(eflo_ref), pack(efhi_ref)], axis=-1)

    @pl.when(pl.program_id(0) < _PROJ_STEPS)
    def _():
        x = x_ref[...]
        pf_ref[...] = jnp.dot(x, w_ref[0:d, :], preferred_element_type=jnp.float32)
        pt_ref[...] = jnp.dot(
            x, w_ref[d : 2 * d, :], preferred_element_type=jnp.float32
        )


_PROJ_STEPS = 5


def _prologue(edge_features, node_features, msg_W, msg_b):
    e, de = edge_features.shape
    n, d = node_features.shape
    dout = msg_W.shape[1]
    blk = 1600
    grid = (e // 2) // blk
    nblk = n // _PROJ_STEPS
    assert grid >= _PROJ_STEPS
    hoff = (e // 2) // blk

    def clamp(i):
        return jnp.minimum(i, _PROJ_STEPS - 1)

    return pl.pallas_call(
        _prologue_body,
        grid=(grid,),
        in_specs=[
            pl.BlockSpec((blk, de), lambda i: (i, 0)),
            pl.BlockSpec((blk, de), lambda i: (i + hoff, 0)),
            pl.BlockSpec((nblk, d), lambda i: (clamp(i), 0)),
            pl.BlockSpec(msg_W.shape, lambda i: (0, 0)),
            pl.BlockSpec((1, dout), lambda i: (0, 0)),
        ],
        out_specs=[
            pl.BlockSpec((blk, dout), lambda i: (i, 0)),
            pl.BlockSpec((nblk, dout), lambda i: (clamp(i), 0)),
            pl.BlockSpec((nblk, dout), lambda i: (clamp(i), 0)),
        ],
        out_shape=[
            jax.ShapeDtypeStruct((e // 2, dout), jnp.int32),
            jax.ShapeDtypeStruct((n, dout), jnp.float32),
            jax.ShapeDtypeStruct((n, dout), jnp.float32),
        ],
    )(edge_features, edge_features, node_features, msg_W, msg_b.reshape(1, dout))


def _mlp_body(agg_ref, x_ref, w1_ref, b1_ref, w2_ref, b2_ref, out_ref):
    agg = agg_ref[0] + agg_ref[1]
    x = x_ref[...]
    d = x.shape[-1]
    h = jnp.maximum(
        jnp.dot(agg, w1_ref[0:d, :], preferred_element_type=jnp.float32)
        + jnp.dot(x, w1_ref[d : 2 * d, :], preferred_element_type=jnp.float32)
        + b1_ref[...],
        0.0,
    )
    h = jnp.maximum(
        jnp.dot(h, w2_ref[...], preferred_element_type=jnp.float32) + b2_ref[...],
        0.0,
    )
    out_ref[...] = x + h


def _node_update(agg_partials, node_features, mlp_W1, mlp_b1, mlp_W2, mlp_b2):
    n, d = node_features.shape
    blk = 2000
    grid = n // blk
    return pl.pallas_call(
        _mlp_body,
        grid=(grid,),
        in_specs=[
            pl.BlockSpec((2, blk, d), lambda i: (0, i, 0)),
            pl.BlockSpec((blk, d), lambda i: (i, 0)),
            pl.BlockSpec(mlp_W1.shape, lambda i: (0, 0)),
            pl.BlockSpec((1, d), lambda i: (0, 0)),
            pl.BlockSpec(mlp_W2.shape, lambda i: (0, 0)),
            pl.BlockSpec((1, d), lambda i: (0, 0)),
        ],
        out_specs=pl.BlockSpec((blk, d), lambda i: (i, 0)),
        out_shape=jax.ShapeDtypeStruct((n, d), jnp.float32),
    )(
        agg_partials,
        node_features,
        mlp_W1,
        mlp_b1.reshape(1, d),
        mlp_W2,
        mlp_b2.reshape(1, d),
    )


_NC = 2
_NS = 16
_NW = _NC * _NS
_B = 40
_CHUNK = 2000
_L = 16


def _sc_body(
    pf_hbm,
    pt_hbm,
    ep_hbm,
    fidx_hbm,
    tidx_hbm,
    zeros_hbm,
    out_hbm,
    acc_sh,
    *slot_refs,
):
    d = pf_hbm.shape[1]
    dw = d // 2
    n_pad = zeros_hbm.shape[0]
    e = fidx_hbm.shape[0]
    ept = e // _NW
    nblocks = ept // _B
    rows = n_pad // _NS

    cid = lax.axis_index("c")
    sid = lax.axis_index("s")
    wid = sid * _NC + cid

    row0 = sid * rows
    pltpu.sync_copy(zeros_hbm.at[pl.ds(row0, rows), :], acc_sh.at[pl.ds(row0, rows), :])
    plsc.subcore_barrier()

    base0 = wid * ept
    half_e = e // 2
    ep0 = lax.rem(base0, half_e)
    ch = (base0 // half_e) * dw

    slots = [tuple(slot_refs[k * 10 : (k + 1) * 10]) for k in range(3)]

    def issue_idx(i, slot):
        fidx, tidx, _fr, _tr, _ep, semi, *_ = slot
        base = base0 + i * _B
        pltpu.async_copy(fidx_hbm.at[pl.ds(base, _B)], fidx, semi)
        pltpu.async_copy(tidx_hbm.at[pl.ds(base, _B)], tidx, semi)

    def wait_idx(i, slot):
        fidx, tidx, _fr, _tr, _ep, semi, *_ = slot
        base = base0 + i * _B
        pltpu.make_async_copy(fidx_hbm.at[pl.ds(base, _B)], fidx, semi).wait()
        pltpu.make_async_copy(tidx_hbm.at[pl.ds(base, _B)], tidx, semi).wait()

    def issue_gathers(i, slot):
        fidx, tidx, fr, tr, ep, _semi, semf, semt, seme, _sems = slot
        erow = pl.multiple_of(ep0 + i * _B, 8)
        pltpu.async_copy(pf_hbm.at[fidx], fr, semf)
        pltpu.async_copy(pt_hbm.at[tidx], tr, semt)
        pltpu.async_copy(ep_hbm.at[pl.ds(erow, _B), :], ep, seme)

    def wait_scatter(slot):
        _fidx, tidx, fr, _tr, _ep, _semi, _semf, _semt, _seme, sems = slot
        pltpu.make_async_copy(fr, acc_sh.at[tidx], sems).wait()

    himask = jnp.full((_L,), -65536, jnp.int32)
    sixteen = jnp.full((_L,), 16, jnp.int32)

    def lo_f32(w):
        return lax.bitcast_convert_type(jnp.left_shift(w, sixteen), jnp.float32)

    def hi_f32(w):
        return lax.bitcast_convert_type(jnp.bitwise_and(w, himask), jnp.float32)

    def process(i, slot):
        fidx, tidx, fr, tr, ep, _semi, semf, semt, seme, sems = slot
        base = base0 + i * _B
        pltpu.make_async_copy(pf_hbm.at[fidx], fr, semf).wait()
        pltpu.make_async_copy(pt_hbm.at[tidx], tr, semt).wait()
        pltpu.make_async_copy(ep_hbm.at[pl.ds(base, _B), :], ep, seme).wait()

        def row(r2, c2):
            for u in range(2):
                r = 2 * r2 + u
                for g in range(dw // _L):
                    we = ep[r, pl.ds(ch + g * _L, _L)]
                    slo = pl.ds(g * _L, _L)
                    shi = pl.ds(dw + g * _L, _L)
                    mlo = fr[r, slo] + tr[r, slo] + lo_f32(we)
                    mhi = fr[r, shi] + tr[r, shi] + hi_f32(we)
                    fr[r, slo] = jnp.maximum(mlo, 0.0)
                    fr[r, shi] = jnp.maximum(mhi, 0.0)
            return c2

        lax.fori_loop(0, _B // 2, row, 0)
        pltpu.async_copy(fr, acc_sh.at[tidx], sems, add=True)

    def step(i, k, first=False, want_gather=True, want_idx=True):
        s, s1, s2 = slots[k], slots[(k + 1) % 3], slots[(k + 2) % 3]
        if want_gather:
            wait_idx(i + 1, s1)
            issue_gathers(i + 1, s1)
        process(i, s)
        if not first:
            wait_scatter(s2)
        if want_idx:
            issue_idx(i + 2, s2)

    issue_idx(0, slots[0])
    issue_idx(1, slots[1])
    wait_idx(0, slots[0])
    issue_gathers(0, slots[0])
    step(0, 0, first=True)

    def triple(g, c2):
        i = 3 * g + 1
        step(i, 1)
        step(i + 1, 2)
        step(i + 2, 0)
        return c2

    lax.fori_loop(0, (nblocks - 4) // 3, triple, 0)

    step(nblocks - 3, (nblocks - 3) % 3)
    step(nblocks - 2, (nblocks - 2) % 3, want_idx=False)
    step(nblocks - 1, (nblocks - 1) % 3, want_gather=False, want_idx=False)
    wait_scatter(slots[(nblocks - 1) % 3])

    plsc.subcore_barrier()
    pltpu.sync_copy(
        acc_sh.at[pl.ds(row0, rows), :], out_hbm.at[cid, pl.ds(row0, rows), :]
    )


def _sc_aggregate(p_from, p_to, eproj, from_idx, to_idx, zeros):
    d = p_from.shape[1]
    dw = d // 2
    n_pad = zeros.shape[0]
    mesh = plsc.VectorSubcoreMesh(core_axis_name="c", subcore_axis_name="s")
    slot = [
        pltpu.VMEM((_B,), jnp.int32),
        pltpu.VMEM((_B,), jnp.int32),
        pltpu.VMEM((_B, d), jnp.float32),
        pltpu.VMEM((_B, d), jnp.float32),
        pltpu.VMEM((_B, d), jnp.int32),
        pltpu.SemaphoreType.DMA,
        pltpu.SemaphoreType.DMA,
        pltpu.SemaphoreType.DMA,
        pltpu.SemaphoreType.DMA,
        pltpu.SemaphoreType.DMA,
    ]
    kern = functools.partial(
        pl.kernel,
        out_type=jax.ShapeDtypeStruct((_NC, n_pad, d), jnp.float32),
        mesh=mesh,
        scratch_types=[pltpu.VMEM_SHARED((n_pad, d), jnp.float32)] + slot * 3,
    )(_sc_body)
    return kern(p_from, p_to, eproj, from_idx, to_idx, zeros)


def kernel(
    node_features,
    from_idx,
    to_idx,
    edge_features,
    msg_W,
    msg_b,
    mlp_W1,
    mlp_b1,
    mlp_W2,
    mlp_b2,
):
    n, d = node_features.shape
    eproj, p_from, p_to = _prologue(edge_features, node_features, msg_W, msg_b)
    n_pad = -(-n // (8 * _NS)) * (8 * _NS)
    zeros = jnp.zeros((n_pad, d), jnp.float32)
    agg_partials = _sc_aggregate(p_from, p_to, eproj, from_idx, to_idx, zeros)
    return _node_update(agg_partials, node_features, mlp_W1, mlp_b1, mlp_W2, mlp_b2)


# speedup vs baseline: 1.0576x; 1.0576x over previous
import functools

import jax
import jax.numpy as jnp
from jax import lax
from jax.experimental import pallas as pl
from jax.experimental.pallas import tpu as pltpu
from jax.experimental.pallas import tpu_sc as plsc


def _prologue_body(ef_ref, x_ref, w_ref, b_ref, out_ref, pf_ref):
    de = ef_ref.shape[-1]
    d = x_ref.shape[-1]
    w = w_ref[w_ref.shape[0] - de :, :]
    ep = jnp.dot(ef_ref[...], w, preferred_element_type=jnp.float32) + b_ref[...]
    bits = lax.bitcast_convert_type(ep.astype(jnp.bfloat16), jnp.uint16)
    half = ep.shape[-1] // 2
    lo = bits[:, :half].astype(jnp.uint32)
    hi = bits[:, half:].astype(jnp.uint32)
    out_ref[...] = lax.bitcast_convert_type(lo | (hi << 16), jnp.int32)

    @pl.when(pl.program_id(0) < _PROJ_STEPS)
    def _():
        x = x_ref[...]
        pf_ref[0] = jnp.dot(x, w_ref[0:d, :], preferred_element_type=jnp.float32)
        pf_ref[1] = jnp.dot(x, w_ref[d : 2 * d, :], preferred_element_type=jnp.float32)


_PROJ_STEPS = 5


def _prologue(edge_features, node_features, msg_W, msg_b):
    e, de = edge_features.shape
    n, d = node_features.shape
    dout = msg_W.shape[1]
    blk = 3200
    grid = e // blk
    nblk = n // _PROJ_STEPS
    assert grid >= _PROJ_STEPS

    def clamp(i):
        return jnp.minimum(i, _PROJ_STEPS - 1)

    return pl.pallas_call(
        _prologue_body,
        grid=(grid,),
        in_specs=[
            pl.BlockSpec((blk, de), lambda i: (i, 0)),
            pl.BlockSpec((nblk, d), lambda i: (clamp(i), 0)),
            pl.BlockSpec(msg_W.shape, lambda i: (0, 0)),
            pl.BlockSpec((1, dout), lambda i: (0, 0)),
        ],
        out_specs=[
            pl.BlockSpec((blk, dout // 2), lambda i: (i, 0)),
            pl.BlockSpec((2, nblk, dout), lambda i: (0, clamp(i), 0)),
        ],
        out_shape=[
            jax.ShapeDtypeStruct((e, dout // 2), jnp.int32),
            jax.ShapeDtypeStruct((2, n, dout), jnp.float32),
        ],
    )(edge_features, node_features, msg_W, msg_b.reshape(1, dout))


def _mlp_body(agg_ref, x_ref, w1_ref, b1_ref, w2_ref, b2_ref, out_ref):
    agg = agg_ref[0] + agg_ref[1]
    x = x_ref[...]
    d = x.shape[-1]
    h = jnp.maximum(
        jnp.dot(agg, w1_ref[0:d, :], preferred_element_type=jnp.float32)
        + jnp.dot(x, w1_ref[d : 2 * d, :], preferred_element_type=jnp.float32)
        + b1_ref[...],
        0.0,
    )
    h = jnp.maximum(
        jnp.dot(h, w2_ref[...], preferred_element_type=jnp.float32) + b2_ref[...],
        0.0,
    )
    out_ref[...] = x + h


def _node_update(agg_partials, node_features, mlp_W1, mlp_b1, mlp_W2, mlp_b2):
    n, d = node_features.shape
    blk = 2000
    grid = n // blk
    return pl.pallas_call(
        _mlp_body,
        grid=(grid,),
        in_specs=[
            pl.BlockSpec((2, blk, d), lambda i: (0, i, 0)),
            pl.BlockSpec((blk, d), lambda i: (i, 0)),
            pl.BlockSpec(mlp_W1.shape, lambda i: (0, 0)),
            pl.BlockSpec((1, d), lambda i: (0, 0)),
            pl.BlockSpec(mlp_W2.shape, lambda i: (0, 0)),
            pl.BlockSpec((1, d), lambda i: (0, 0)),
        ],
        out_specs=pl.BlockSpec((blk, d), lambda i: (i, 0)),
        out_shape=jax.ShapeDtypeStruct((n, d), jnp.float32),
    )(
        agg_partials,
        node_features,
        mlp_W1,
        mlp_b1.reshape(1, d),
        mlp_W2,
        mlp_b2.reshape(1, d),
    )


_NC = 2
_NS = 16
_NW = _NC * _NS
_B = 40
_CHUNK = 2000
_L = 16


def _sc_body(
    t_hbm,
    ep_hbm,
    cidx_hbm,
    tidx_hbm,
    zeros_hbm,
    out_hbm,
    acc_sh,
    *slot_refs,
):
    d = t_hbm.shape[1]
    dw = d // 2
    n_pad = zeros_hbm.shape[0]
    e = tidx_hbm.shape[0]
    ept = e // _NW
    nblocks = ept // _B
    rows = n_pad // _NS

    cid = lax.axis_index("c")
    sid = lax.axis_index("s")
    wid = sid * _NC + cid

    row0 = sid * rows
    pltpu.sync_copy(zeros_hbm.at[pl.ds(row0, rows), :], acc_sh.at[pl.ds(row0, rows), :])
    plsc.subcore_barrier()

    base0 = wid * ept

    slots = [tuple(slot_refs[k * 8 : (k + 1) * 8]) for k in range(3)]

    def issue_idx(i, slot):
        cidx, tidx, _frt, _ep, semi, *_ = slot
        base = base0 + i * _B
        pltpu.async_copy(cidx_hbm.at[pl.ds(2 * base, 2 * _B)], cidx, semi)
        pltpu.async_copy(tidx_hbm.at[pl.ds(base, _B)], tidx, semi)

    def wait_idx(i, slot):
        cidx, tidx, _frt, _ep, semi, *_ = slot
        base = base0 + i * _B
        pltpu.make_async_copy(cidx_hbm.at[pl.ds(2 * base, 2 * _B)], cidx, semi).wait()
        pltpu.make_async_copy(tidx_hbm.at[pl.ds(base, _B)], tidx, semi).wait()

    def issue_gathers(i, slot):
        cidx, _tidx, frt, ep, _semi, semg, seme, _sems = slot
        base = base0 + i * _B
        pltpu.async_copy(t_hbm.at[cidx], frt, semg)
        pltpu.async_copy(ep_hbm.at[pl.ds(base, _B), :], ep, seme)

    def wait_scatter(slot):
        _cidx, tidx, frt, _ep, _semi, _semg, _seme, sems = slot
        pltpu.make_async_copy(frt.at[pl.ds(0, _B), :], acc_sh.at[tidx], sems).wait()

    himask = jnp.full((_L,), -65536, jnp.int32)
    sixteen = jnp.full((_L,), 16, jnp.int32)

    def lo_f32(w):
        return lax.bitcast_convert_type(jnp.left_shift(w, sixteen), jnp.float32)

    def hi_f32(w):
        return lax.bitcast_convert_type(jnp.bitwise_and(w, himask), jnp.float32)

    def process(i, slot):
        cidx, tidx, frt, ep, _semi, semg, seme, sems = slot
        base = base0 + i * _B
        pltpu.make_async_copy(t_hbm.at[cidx], frt, semg).wait()
        pltpu.make_async_copy(ep_hbm.at[pl.ds(base, _B), :], ep, seme).wait()

        def row(r2, c2):
            for u in range(2):
                r = 2 * r2 + u
                for g in range(dw // _L):
                    we = ep[r, pl.ds(g * _L, _L)]
                    slo = pl.ds(g * _L, _L)
                    shi = pl.ds(dw + g * _L, _L)
                    mlo = frt[r, slo] + frt[_B + r, slo] + lo_f32(we)
                    mhi = frt[r, shi] + frt[_B + r, shi] + hi_f32(we)
                    frt[r, slo] = jnp.maximum(mlo, 0.0)
                    frt[r, shi] = jnp.maximum(mhi, 0.0)
            return c2

        lax.fori_loop(0, _B // 2, row, 0)
        pltpu.async_copy(frt.at[pl.ds(0, _B), :], acc_sh.at[tidx], sems, add=True)

    def step(i, k, first=False, want_gather=True, want_idx=True):
        s, s1, s2 = slots[k], slots[(k + 1) % 3], slots[(k + 2) % 3]
        if want_gather:
            wait_idx(i + 1, s1)
            issue_gathers(i + 1, s1)
        process(i, s)
        if not first:
            wait_scatter(s2)
        if want_idx:
            issue_idx(i + 2, s2)

    issue_idx(0, slots[0])
    issue_idx(1, slots[1])
    wait_idx(0, slots[0])
    issue_gathers(0, slots[0])
    step(0, 0, first=True)

    def triple(g, c2):
        i = 3 * g + 1
        step(i, 1)
        step(i + 1, 2)
        step(i + 2, 0)
        return c2

    lax.fori_loop(0, (nblocks - 4) // 3, triple, 0)

    step(nblocks - 3, (nblocks - 3) % 3)
    step(nblocks - 2, (nblocks - 2) % 3, want_idx=False)
    step(nblocks - 1, (nblocks - 1) % 3, want_gather=False, want_idx=False)
    wait_scatter(slots[(nblocks - 1) % 3])

    plsc.subcore_barrier()
    pltpu.sync_copy(
        acc_sh.at[pl.ds(row0, rows), :], out_hbm.at[cid, pl.ds(row0, rows), :]
    )


def _sc_aggregate(table, eproj, cat_idx, to_idx, zeros):
    d = table.shape[1]
    dw = d // 2
    n_pad = zeros.shape[0]
    mesh = plsc.VectorSubcoreMesh(core_axis_name="c", subcore_axis_name="s")
    slot = [
        pltpu.VMEM((2 * _B,), jnp.int32),
        pltpu.VMEM((_B,), jnp.int32),
        pltpu.VMEM((2 * _B, d), jnp.float32),
        pltpu.VMEM((_B, dw), jnp.int32),
        pltpu.SemaphoreType.DMA,
        pltpu.SemaphoreType.DMA,
        pltpu.SemaphoreType.DMA,
        pltpu.SemaphoreType.DMA,
    ]
    kern = functools.partial(
        pl.kernel,
        out_type=jax.ShapeDtypeStruct((_NC, n_pad, d), jnp.float32),
        mesh=mesh,
        scratch_types=[pltpu.VMEM_SHARED((n_pad, d), jnp.float32)] + slot * 3,
    )(_sc_body)
    return kern(table, eproj, cat_idx, to_idx, zeros)


def kernel(
    node_features,
    from_idx,
    to_idx,
    edge_features,
    msg_W,
    msg_b,
    mlp_W1,
    mlp_b1,
    mlp_W2,
    mlp_b2,
):
    n, d = node_features.shape
    eproj, table = _prologue(edge_features, node_features, msg_W, msg_b)
    table = table.reshape(2 * n, d)
    cat_idx = jnp.stack(
        [from_idx.reshape(-1, _B), to_idx.reshape(-1, _B) + n], axis=1
    ).reshape(-1)
    n_pad = -(-n // (8 * _NS)) * (8 * _NS)
    zeros = jnp.zeros((n_pad, d), jnp.float32)
    agg_partials = _sc_aggregate(table, eproj, cat_idx, to_idx, zeros)
    return _node_update(agg_partials, node_features, mlp_W1, mlp_b1, mlp_W2, mlp_b2)


# speedup vs baseline: 1.1726x; 1.1087x over previous
import functools

import jax
import jax.numpy as jnp
from jax import lax
from jax.experimental import pallas as pl
from jax.experimental.pallas import tpu as pltpu
from jax.experimental.pallas import tpu_sc as plsc


def _prologue_body(ef_ref, x_ref, w_ref, b_ref, out_ref, pf_ref, pt_ref):
    de = ef_ref.shape[-1]
    d = x_ref.shape[-1]
    w = w_ref[w_ref.shape[0] - de :, :]
    ep = jnp.dot(ef_ref[...], w, preferred_element_type=jnp.float32) + b_ref[...]
    bits = lax.bitcast_convert_type(ep.astype(jnp.bfloat16), jnp.uint16)
    half = ep.shape[-1] // 2
    lo = bits[:, :half].astype(jnp.uint32)
    hi = bits[:, half:].astype(jnp.uint32)
    out_ref[...] = lax.bitcast_convert_type(lo | (hi << 16), jnp.int32)

    @pl.when(pl.program_id(0) < _PROJ_STEPS)
    def _():
        x = x_ref[...]
        pf_ref[...] = jnp.dot(x, w_ref[0:d, :], preferred_element_type=jnp.float32)
        pt_ref[...] = jnp.dot(
            x, w_ref[d : 2 * d, :], preferred_element_type=jnp.float32
        )


_PROJ_STEPS = 5


def _prologue(edge_features, node_features, msg_W, msg_b):
    e, de = edge_features.shape
    n, d = node_features.shape
    dout = msg_W.shape[1]
    blk = 3200
    grid = e // blk
    nblk = n // _PROJ_STEPS
    assert grid >= _PROJ_STEPS

    def clamp(i):
        return jnp.minimum(i, _PROJ_STEPS - 1)

    return pl.pallas_call(
        _prologue_body,
        grid=(grid,),
        in_specs=[
            pl.BlockSpec((blk, de), lambda i: (i, 0)),
            pl.BlockSpec((nblk, d), lambda i: (clamp(i), 0)),
            pl.BlockSpec(msg_W.shape, lambda i: (0, 0)),
            pl.BlockSpec((1, dout), lambda i: (0, 0)),
        ],
        out_specs=[
            pl.BlockSpec((blk, dout // 2), lambda i: (i, 0)),
            pl.BlockSpec((nblk, dout), lambda i: (clamp(i), 0)),
            pl.BlockSpec((nblk, dout), lambda i: (clamp(i), 0)),
        ],
        out_shape=[
            jax.ShapeDtypeStruct((e, dout // 2), jnp.int32),
            jax.ShapeDtypeStruct((n, dout), jnp.float32),
            jax.ShapeDtypeStruct((n, dout), jnp.float32),
        ],
    )(edge_features, node_features, msg_W, msg_b.reshape(1, dout))


def _mlp_body(agg_ref, x_ref, w1_ref, b1_ref, w2_ref, b2_ref, out_ref):
    agg = agg_ref[0] + agg_ref[1]
    x = x_ref[...]
    d = x.shape[-1]
    h = jnp.maximum(
        jnp.dot(agg, w1_ref[0:d, :], preferred_element_type=jnp.float32)
        + jnp.dot(x, w1_ref[d : 2 * d, :], preferred_element_type=jnp.float32)
        + b1_ref[...],
        0.0,
    )
    h = jnp.maximum(
        jnp.dot(h, w2_ref[...], preferred_element_type=jnp.float32) + b2_ref[...],
        0.0,
    )
    out_ref[...] = x + h


def _node_update(agg_partials, node_features, mlp_W1, mlp_b1, mlp_W2, mlp_b2):
    n, d = node_features.shape
    blk = 2000
    grid = n // blk
    return pl.pallas_call(
        _mlp_body,
        grid=(grid,),
        in_specs=[
            pl.BlockSpec((2, blk, d), lambda i: (0, i, 0)),
            pl.BlockSpec((blk, d), lambda i: (i, 0)),
            pl.BlockSpec(mlp_W1.shape, lambda i: (0, 0)),
            pl.BlockSpec((1, d), lambda i: (0, 0)),
            pl.BlockSpec(mlp_W2.shape, lambda i: (0, 0)),
            pl.BlockSpec((1, d), lambda i: (0, 0)),
        ],
        out_specs=pl.BlockSpec((blk, d), lambda i: (i, 0)),
        out_shape=jax.ShapeDtypeStruct((n, d), jnp.float32),
    )(
        agg_partials,
        node_features,
        mlp_W1,
        mlp_b1.reshape(1, d),
        mlp_W2,
        mlp_b2.reshape(1, d),
    )


_NC = 2
_NS = 16
_NW = _NC * _NS
_B = 40
_CHUNK = 2000
_L = 16


def _sc_body(
    pf_hbm,
    pt_hbm,
    ep_hbm,
    fidx_hbm,
    tidx_hbm,
    zeros_hbm,
    out_hbm,
    acc_sh,
    *slot_refs,
):
    d = pf_hbm.shape[1]
    dw = d // 2
    n_pad = zeros_hbm.shape[0]
    e = fidx_hbm.shape[0]
    ept = e // _NW
    nblocks = ept // _B
    rows = n_pad // _NS

    cid = lax.axis_index("c")
    sid = lax.axis_index("s")
    wid = sid * _NC + cid

    row0 = sid * rows
    pltpu.sync_copy(zeros_hbm.at[pl.ds(row0, rows), :], acc_sh.at[pl.ds(row0, rows), :])
    plsc.subcore_barrier()

    base0 = wid * ept

    slots = [tuple(slot_refs[k * 10 : (k + 1) * 10]) for k in range(3)]

    def issue_idx(i, slot):
        fidx, tidx, _fr, _tr, _ep, semi, *_ = slot
        base = base0 + i * _B
        pltpu.async_copy(fidx_hbm.at[pl.ds(base, _B)], fidx, semi)
        pltpu.async_copy(tidx_hbm.at[pl.ds(base, _B)], tidx, semi)

    def wait_idx(i, slot):
        fidx, tidx, _fr, _tr, _ep, semi, *_ = slot
        base = base0 + i * _B
        pltpu.make_async_copy(fidx_hbm.at[pl.ds(base, _B)], fidx, semi).wait()
        pltpu.make_async_copy(tidx_hbm.at[pl.ds(base, _B)], tidx, semi).wait()

    def issue_gathers(i, slot):
        fidx, tidx, fr, tr, ep, _semi, semf, semt, seme, _sems = slot
        base = base0 + i * _B
        pltpu.async_copy(pf_hbm.at[fidx], fr, semf)
        pltpu.async_copy(pt_hbm.at[tidx], tr, semt)
        pltpu.async_copy(ep_hbm.at[pl.ds(base, _B), :], ep, seme)

    def wait_scatter(slot):
        _fidx, tidx, fr, _tr, _ep, _semi, _semf, _semt, _seme, sems = slot
        pltpu.make_async_copy(fr, acc_sh.at[tidx], sems).wait()

    himask = jnp.full((_L,), -65536, jnp.int32)
    sixteen = jnp.full((_L,), 16, jnp.int32)

    def lo_f32(w):
        return lax.bitcast_convert_type(jnp.left_shift(w, sixteen), jnp.float32)

    def hi_f32(w):
        return lax.bitcast_convert_type(jnp.bitwise_and(w, himask), jnp.float32)

    def process(i, slot):
        fidx, tidx, fr, tr, ep, _semi, semf, semt, seme, sems = slot
        base = base0 + i * _B
        pltpu.make_async_copy(pf_hbm.at[fidx], fr, semf).wait()
        pltpu.make_async_copy(pt_hbm.at[tidx], tr, semt).wait()
        pltpu.make_async_copy(ep_hbm.at[pl.ds(base, _B), :], ep, seme).wait()

        def row(r2, c2):
            for u in range(2):
                r = 2 * r2 + u
                for g in range(dw // _L):
                    we = ep[r, pl.ds(g * _L, _L)]
                    slo = pl.ds(g * _L, _L)
                    shi = pl.ds(dw + g * _L, _L)
                    mlo = fr[r, slo] + tr[r, slo] + lo_f32(we)
                    mhi = fr[r, shi] + tr[r, shi] + hi_f32(we)
                    fr[r, slo] = jnp.maximum(mlo, 0.0)
                    fr[r, shi] = jnp.maximum(mhi, 0.0)
            return c2

        lax.fori_loop(0, _B // 2, row, 0)
        pltpu.async_copy(fr, acc_sh.at[tidx], sems, add=True)

    def step(i, k, first=False, want_gather=True, want_idx=True):
        s, s1, s2 = slots[k], slots[(k + 1) % 3], slots[(k + 2) % 3]
        if want_gather:
            wait_idx(i + 1, s1)
            issue_gathers(i + 1, s1)
        process(i, s)
        if not first:
            wait_scatter(s2)
        if want_idx:
            issue_idx(i + 2, s2)

    issue_idx(0, slots[0])
    issue_idx(1, slots[1])
    wait_idx(0, slots[0])
    issue_gathers(0, slots[0])
    step(0, 0, first=True)

    def triple(g, c2):
        i = 3 * g + 1
        step(i, 1)
        step(i + 1, 2)
        step(i + 2, 0)
        return c2

    lax.fori_loop(0, (nblocks - 4) // 3, triple, 0)

    step(nblocks - 3, (nblocks - 3) % 3)
    step(nblocks - 2, (nblocks - 2) % 3, want_idx=False)
    step(nblocks - 1, (nblocks - 1) % 3, want_gather=False, want_idx=False)
    wait_scatter(slots[(nblocks - 1) % 3])

    plsc.subcore_barrier()
    pltpu.sync_copy(
        acc_sh.at[pl.ds(row0, rows), :], out_hbm.at[cid, pl.ds(row0, rows), :]
    )


def _sc_aggregate(p_from, p_to, eproj, from_idx, to_idx, zeros):
    d = p_from.shape[1]
    dw = d // 2
    n_pad = zeros.shape[0]
    mesh = plsc.VectorSubcoreMesh(core_axis_name="c", subcore_axis_name="s")
    slot = [
        pltpu.VMEM((_B,), jnp.int32),
        pltpu.VMEM((_B,), jnp.int32),
        pltpu.VMEM((_B, d), jnp.float32),
        pltpu.VMEM((_B, d), jnp.float32),
        pltpu.VMEM((_B, dw), jnp.int32),
        pltpu.SemaphoreType.DMA,
        pltpu.SemaphoreType.DMA,
        pltpu.SemaphoreType.DMA,
        pltpu.SemaphoreType.DMA,
        pltpu.SemaphoreType.DMA,
    ]
    kern = functools.partial(
        pl.kernel,
        out_type=jax.ShapeDtypeStruct((_NC, n_pad, d), jnp.float32),
        mesh=mesh,
        scratch_types=[pltpu.VMEM_SHARED((n_pad, d), jnp.float32)] + slot * 3,
    )(_sc_body)
    return kern(p_from, p_to, eproj, from_idx, to_idx, zeros)


def kernel(
    node_features,
    from_idx,
    to_idx,
    edge_features,
    msg_W,
    msg_b,
    mlp_W1,
    mlp_b1,
    mlp_W2,
    mlp_b2,
):
    n, d = node_features.shape
    eproj, p_from, p_to = _prologue(edge_features, node_features, msg_W, msg_b)
    n_pad = -(-n // (8 * _NS)) * (8 * _NS)
    zeros = jnp.zeros((n_pad, d), jnp.float32)
    agg_partials = _sc_aggregate(p_from, p_to, eproj, from_idx, to_idx, zeros)
    return _node_update(agg_partials, node_features, mlp_W1, mlp_b1, mlp_W2, mlp_b2)
